# baseline (speedup 1.0000x reference)
import jax
import jax.numpy as jnp
from jax.experimental import pallas as pl
from jax.experimental.pallas import tpu as pltpu

_B = 4096
_D = 64
_NE = 100000
_IDX_PAD = 512
_M_TILE = 32
_N_STEPS = _B // _M_TILE


def _hr_kernel(data_ref, ehead_ref, rel_ref, hr_ref):
    idx_h = data_ref[:, 0:1]
    idx_r = data_ref[:, 1:2]
    cols = jax.lax.broadcasted_iota(jnp.int32, (_B, _IDX_PAD), 1)
    oh_h = (idx_h == cols).astype(jnp.float32)
    oh_r = (idx_r == cols).astype(jnp.float32)
    h = jnp.dot(oh_h, ehead_ref[...], preferred_element_type=jnp.float32)
    r = jnp.dot(oh_r, rel_ref[...], preferred_element_type=jnp.float32)
    hr_ref[...] = (h * r).astype(jnp.bfloat16)


def _score_kernel(hr_ref, e_ref, out_ref):
    score = jax.lax.dot_general(
        hr_ref[...], e_ref[...],
        (((1,), (1,)), ((), ())),
        preferred_element_type=jnp.float32,
    )
    out_ref[...] = jax.nn.sigmoid(score)


def kernel(triple_hop1, triple_hop2, data, entity_embed, relation_embed):
    del triple_hop1, triple_hop2
    ehead = entity_embed[:_IDX_PAD]
    rel = jnp.pad(relation_embed, ((0, _IDX_PAD - relation_embed.shape[0]), (0, 0)))
    hr = pl.pallas_call(
        _hr_kernel,
        out_shape=jax.ShapeDtypeStruct((_B, _D), jnp.bfloat16),
    )(data, ehead, rel)
    e_bf16 = entity_embed.astype(jnp.bfloat16)
    out = pl.pallas_call(
        _score_kernel,
        grid=(_N_STEPS,),
        in_specs=[
            pl.BlockSpec((_M_TILE, _D), lambda i: (i, 0)),
            pl.BlockSpec((_NE, _D), lambda i: (0, 0)),
        ],
        out_specs=pl.BlockSpec((_M_TILE, _NE), lambda i: (i, 0)),
        out_shape=jax.ShapeDtypeStruct((_B, _NE), jnp.float32),
        compiler_params=pltpu.CompilerParams(
            dimension_semantics=("arbitrary",),
        ),
    )(hr, e_bf16)
    return out


# speedup vs baseline: 4.3517x; 4.3517x over previous
import jax
import jax.numpy as jnp
from jax.experimental import pallas as pl
from jax.experimental.pallas import tpu as pltpu

_B = 4096
_D = 64
_NE = 100000
_IDX_PAD = 512
_E_TILE = 1000
_N_STEPS = _NE // _E_TILE


def _hrt_kernel(datat_ref, eheadt_ref, relt_ref, hrt_ref):
    idx_h = datat_ref[0:1, :]
    idx_r = datat_ref[1:2, :]
    rows = jax.lax.broadcasted_iota(jnp.int32, (_IDX_PAD, _B), 0)
    oht_h = (rows == idx_h).astype(jnp.float32)
    oht_r = (rows == idx_r).astype(jnp.float32)
    ht = jnp.dot(eheadt_ref[...], oht_h, preferred_element_type=jnp.float32)
    rt = jnp.dot(relt_ref[...], oht_r, preferred_element_type=jnp.float32)
    hrt_ref[...] = (ht * rt).astype(jnp.bfloat16)


def _score_kernel(e_ref, hrt_ref, out_ref):
    score_t = jax.lax.dot_general(
        e_ref[...], hrt_ref[...],
        (((1,), (0,)), ((), ())),
        preferred_element_type=jnp.float32,
    )
    out_ref[...] = jax.nn.sigmoid(score_t)


def kernel(triple_hop1, triple_hop2, data, entity_embed, relation_embed):
    del triple_hop1, triple_hop2
    datat = jnp.pad(data.T, ((0, 5), (0, 0)))
    eheadt = entity_embed[:_IDX_PAD].T
    relt = jnp.pad(relation_embed, ((0, _IDX_PAD - relation_embed.shape[0]), (0, 0))).T
    hrt = pl.pallas_call(
        _hrt_kernel,
        out_shape=jax.ShapeDtypeStruct((_D, _B), jnp.bfloat16),
    )(datat, eheadt, relt)
    e_bf16 = entity_embed.astype(jnp.bfloat16)
    score_t = pl.pallas_call(
        _score_kernel,
        grid=(_N_STEPS,),
        in_specs=[
            pl.BlockSpec((_E_TILE, _D), lambda i: (i, 0)),
            pl.BlockSpec((_D, _B), lambda i: (0, 0)),
        ],
        out_specs=pl.BlockSpec((_E_TILE, _B), lambda i: (i, 0)),
        out_shape=jax.ShapeDtypeStruct((_NE, _B), jnp.float32),
        compiler_params=pltpu.CompilerParams(
            dimension_semantics=("arbitrary",),
        ),
    )(e_bf16, hrt)
    return score_t.T
